# baseline (speedup 1.0000x reference)
import dataclasses
import functools

import jax
import jax.numpy as jnp
from jax import lax
from jax.experimental import pallas as pl
from jax.experimental.pallas import tpu as pltpu
from jax.experimental.pallas import tpu_sc as plsc

_NC, _NS = 2, 16
_NW = _NC * _NS


def kernel(x, table):
    B, T = x.shape
    V, D = table.shape
    idxT = x.T.astype(jnp.int32)
    big = jnp.pad(table, ((0, 0), (0, 128 - D)))

    b_per = B // _NW
    assert b_per == 128 and T % 2 == 0

    mesh = plsc.VectorSubcoreMesh(core_axis_name="c", subcore_axis_name="s")
    cp = pltpu.CompilerParams()
    if "needs_layout_passes" in pltpu.CompilerParams.__dataclass_fields__:
        cp = dataclasses.replace(cp, needs_layout_passes=False)

    @functools.partial(
        pl.kernel,
        out_type=jax.ShapeDtypeStruct((T, D, B), jnp.float32),
        mesh=mesh,
        compiler_params=cp,
        scratch_types=[
            pltpu.VMEM((T, b_per), jnp.int32),
            pltpu.VMEM((b_per, 128), jnp.float32),
            pltpu.VMEM((b_per, 128), jnp.float32),
            pltpu.VMEM((D, b_per), jnp.float32),
            pltpu.VMEM((D, b_per), jnp.float32),
            pltpu.SemaphoreType.DMA,
            pltpu.SemaphoreType.DMA,
            pltpu.SemaphoreType.DMA,
            pltpu.SemaphoreType.DMA,
        ],
    )
    def _gather(tab_hbm, idx_hbm, out_hbm, idx2d, buf0, buf1, ob0, ob1,
                sg0, sg1, sw0, sw1):
        wid = lax.axis_index("s") * _NC + lax.axis_index("c")
        b0 = wid * b_per
        pltpu.sync_copy(idx_hbm.at[:, pl.ds(b0, b_per)], idx2d)

        rowvs = [jnp.arange(16, dtype=jnp.int32) + 16 * g for g in range(8)]

        def gather(t, buf, sem):
            pltpu.async_copy(tab_hbm.at[idx2d.at[t]], buf, sem)

        def wait_gather(buf, sem):
            pltpu.make_async_copy(tab_hbm.at[idx2d.at[0]], buf, sem).wait()

        def transpose(buf, ob):
            @pl.loop(0, D)
            def _(d):
                colv = jnp.full((16,), d, dtype=jnp.int32)
                for g in range(8):
                    ob.at[d, pl.ds(16 * g, 16)][...] = plsc.load_gather(
                        buf, [rowvs[g], colv])

        def write(t, ob, sem):
            pltpu.async_copy(ob, out_hbm.at[t, :, pl.ds(b0, b_per)], sem)

        def wait_write(ob, sem):
            pltpu.make_async_copy(ob, out_hbm.at[0, :, pl.ds(b0, b_per)], sem).wait()

        gather(0, buf0, sg0)
        gather(1, buf1, sg1)

        @pl.loop(0, T // 2)
        def _(i):
            t = 2 * i
            wait_gather(buf0, sg0)

            @pl.when(i > 0)
            def _():
                wait_write(ob0, sw0)

            transpose(buf0, ob0)

            @pl.when(i < T // 2 - 1)
            def _():
                gather(t + 2, buf0, sg0)

            write(t, ob0, sw0)

            wait_gather(buf1, sg1)

            @pl.when(i > 0)
            def _():
                wait_write(ob1, sw1)

            transpose(buf1, ob1)

            @pl.when(i < T // 2 - 1)
            def _():
                gather(t + 3, buf1, sg1)

            write(t + 1, ob1, sw1)

        wait_write(ob0, sw0)
        wait_write(ob1, sw1)

    out_tdb = _gather(big, idxT)
    return jnp.transpose(out_tdb, (2, 0, 1))


# speedup vs baseline: 1.0992x; 1.0992x over previous
import functools

import jax
import jax.numpy as jnp
from jax import lax
from jax.experimental import pallas as pl
from jax.experimental.pallas import tpu as pltpu
from jax.experimental.pallas import tpu_sc as plsc

_NC, _NS = 2, 16
_NW = _NC * _NS
_W = 200
_BPC = 4


def kernel(x, table):
    B, T = x.shape
    V, D = table.shape
    n = B * T
    idx = x.reshape(n).astype(jnp.int32)
    idx2 = lax.shift_right_logical(idx, 1)
    packed = table.reshape(V // 2, 2 * D)

    n_per = n // _NW
    b_per = B // _NW
    n_chunks = n_per // _W
    assert n_chunks % 2 == 0

    mesh = plsc.VectorSubcoreMesh(core_axis_name="c", subcore_axis_name="s")

    @functools.partial(
        pl.kernel,
        out_type=jax.ShapeDtypeStruct((B, T, D), jnp.float32),
        mesh=mesh,
        scratch_types=[
            pltpu.VMEM((n_per,), jnp.int32),
            pltpu.VMEM((n_per,), jnp.int32),
            pltpu.SMEM((_W,), jnp.int32),
            pltpu.SMEM((_W,), jnp.int32),
            pltpu.VMEM((_W, 128), jnp.float32),
            pltpu.VMEM((_W, 128), jnp.float32),
            pltpu.VMEM((_W, D), jnp.float32),
            pltpu.VMEM((_W, D), jnp.float32),
            pltpu.SemaphoreType.DMA,
            pltpu.SemaphoreType.DMA,
            pltpu.SemaphoreType.DMA,
            pltpu.SemaphoreType.DMA,
        ],
    )
    def _gather(tab_hbm, idx2_hbm, idx_hbm, out_hbm, idx_all, idx_v, is0, is1,
                buf0, buf1, ob0, ob1, sg0, sg1, sw0, sw1):
        wid = lax.axis_index("s") * _NC + lax.axis_index("c")
        base = wid * n_per
        b0 = wid * b_per
        pltpu.sync_copy(idx2_hbm.at[pl.ds(base, n_per)], idx_all)
        pltpu.sync_copy(idx_hbm.at[pl.ds(base, n_per)], idx_v)

        def gather(c, buf, sem):
            pltpu.async_copy(tab_hbm.at[idx_all.at[pl.ds(c * _W, _W)]], buf, sem)

        def wait_gather(buf, sem):
            pltpu.make_async_copy(
                tab_hbm.at[idx_all.at[pl.ds(0, _W)]], buf, sem).wait()

        def repack(c, buf, ob, idx_s):
            del idx_s

            @pl.loop(0, _W)
            def _(r):
                off = (idx_v[pl.ds(c * _W + r, 1)][0] & 1) * D
                for k in range(D // 16):
                    ob.at[pl.ds(r, 1), pl.ds(16 * k, 16)][...] = (
                        buf.at[pl.ds(r, 1), pl.ds(off + 16 * k, 16)][...])

        def write(c, ob, sem):
            bc = b0 + c * _BPC
            for j in range(_BPC):
                pltpu.async_copy(ob.at[pl.ds(T * j, T)], out_hbm.at[bc + j], sem)

        def wait_write(ob, sem):
            for j in range(_BPC):
                pltpu.make_async_copy(ob.at[pl.ds(T * j, T)], out_hbm.at[0], sem).wait()

        gather(0, buf0, sg0)
        gather(1, buf1, sg1)

        @pl.loop(0, n_chunks // 2)
        def _(i):
            c = 2 * i
            wait_gather(buf0, sg0)

            @pl.when(i > 0)
            def _():
                wait_write(ob0, sw0)

            repack(c, buf0, ob0, is0)

            @pl.when(i < n_chunks // 2 - 1)
            def _():
                gather(c + 2, buf0, sg0)

            write(c, ob0, sw0)

            wait_gather(buf1, sg1)

            @pl.when(i > 0)
            def _():
                wait_write(ob1, sw1)

            repack(c + 1, buf1, ob1, is1)

            @pl.when(i < n_chunks // 2 - 1)
            def _():
                gather(c + 3, buf1, sg1)

            write(c + 1, ob1, sw1)

        wait_write(ob0, sw0)
        wait_write(ob1, sw1)

    return _gather(packed, idx2, idx)


# speedup vs baseline: 1.5936x; 1.4498x over previous
import functools

import jax
import jax.numpy as jnp
from jax import lax
from jax.experimental import pallas as pl
from jax.experimental.pallas import tpu as pltpu
from jax.experimental.pallas import tpu_sc as plsc

_NC, _NS = 2, 16
_NW = _NC * _NS
_K = 2
_W = 200


def _make_gather(B, Th, D, V):
    n = B * Th
    n_per = n // _NW
    b_per = B // _NW
    bpc = _W // Th
    n_chunks = n_per // _W
    assert _W % Th == 0 and n_per % _W == 0 and n_chunks % 2 == 0

    mesh = plsc.VectorSubcoreMesh(core_axis_name="c", subcore_axis_name="s")

    @functools.partial(
        pl.kernel,
        out_type=jax.ShapeDtypeStruct((B, Th, D), jnp.float32),
        mesh=mesh,
        scratch_types=[
            pltpu.VMEM((n_per,), jnp.int32),
            pltpu.VMEM((_W, 128), jnp.float32),
            pltpu.VMEM((_W, 128), jnp.float32),
            pltpu.VMEM((_W, D), jnp.float32),
            pltpu.VMEM((_W, D), jnp.float32),
            pltpu.SemaphoreType.DMA,
            pltpu.SemaphoreType.DMA,
            pltpu.SemaphoreType.DMA,
            pltpu.SemaphoreType.DMA,
        ],
    )
    def _gather(tab_hbm, idx_hbm, out_hbm, idx_all, buf0, buf1, ob0, ob1,
                sg0, sg1, sw0, sw1):
        wid = lax.axis_index("s") * _NC + lax.axis_index("c")
        base = wid * n_per
        b0 = wid * b_per
        pltpu.sync_copy(idx_hbm.at[pl.ds(base, n_per)], idx_all)

        def gather(c, buf, sem):
            pltpu.async_copy(tab_hbm.at[idx_all.at[pl.ds(c * _W, _W)]], buf, sem)

        def wait_gather(buf, sem):
            pltpu.make_async_copy(
                tab_hbm.at[idx_all.at[pl.ds(0, _W)]], buf, sem).wait()

        def repack(buf, ob):
            @pl.loop(0, _W)
            def _(r):
                for k in range(D // 16):
                    ob.at[pl.ds(r, 1), pl.ds(16 * k, 16)][...] = (
                        buf.at[pl.ds(r, 1), pl.ds(16 * k, 16)][...])

        def write(c, ob, sem):
            bc = b0 + c * bpc
            for j in range(bpc):
                pltpu.async_copy(ob.at[pl.ds(Th * j, Th)], out_hbm.at[bc + j], sem)

        def wait_write(ob, sem):
            for j in range(bpc):
                pltpu.make_async_copy(ob.at[pl.ds(Th * j, Th)], out_hbm.at[0], sem).wait()

        gather(0, buf0, sg0)
        gather(1, buf1, sg1)

        @pl.loop(0, n_chunks // 2)
        def _(i):
            c = 2 * i
            wait_gather(buf0, sg0)

            @pl.when(i > 0)
            def _():
                wait_write(ob0, sw0)

            repack(buf0, ob0)

            @pl.when(i < n_chunks // 2 - 1)
            def _():
                gather(c + 2, buf0, sg0)

            write(c, ob0, sw0)

            wait_gather(buf1, sg1)

            @pl.when(i > 0)
            def _():
                wait_write(ob1, sw1)

            repack(buf1, ob1)

            @pl.when(i < n_chunks // 2 - 1)
            def _():
                gather(c + 3, buf1, sg1)

            write(c + 1, ob1, sw1)

        wait_write(ob0, sw0)
        wait_write(ob1, sw1)

    return _gather


def kernel(x, table):
    B, T = x.shape
    V, D = table.shape
    big = jnp.pad(table, ((0, 0), (0, 128 - D)))
    Th = T // _K
    gather_half = _make_gather(B, Th, D, V)

    halves = []
    for k in range(_K):
        idxk = x[:, Th * k:Th * (k + 1)].reshape(-1).astype(jnp.int32)
        yk = gather_half(big, idxk)
        halves.append(jnp.transpose(yk, (1, 2, 0)))

    out_tdb = jnp.concatenate(halves, axis=0)
    return jnp.transpose(out_tdb, (2, 0, 1))


# speedup vs baseline: 1.6765x; 1.0520x over previous
import functools

import jax
import jax.numpy as jnp
from jax import lax
from jax.experimental import pallas as pl
from jax.experimental.pallas import tpu as pltpu
from jax.experimental.pallas import tpu_sc as plsc

_NC, _NS = 2, 16
_NW = _NC * _NS
_K = 1
_W = 200


def _make_gather(B, Th, D, V):
    n = B * Th
    n_per = n // _NW
    b_per = B // _NW
    bpc = _W // Th
    n_chunks = n_per // _W
    assert _W % Th == 0 and n_per % _W == 0 and n_chunks % 2 == 0

    mesh = plsc.VectorSubcoreMesh(core_axis_name="c", subcore_axis_name="s")

    @functools.partial(
        pl.kernel,
        out_type=jax.ShapeDtypeStruct((B, Th, D), jnp.float32),
        mesh=mesh,
        scratch_types=[
            pltpu.VMEM((n_per,), jnp.int32),
            pltpu.VMEM((_W, 128), jnp.float32),
            pltpu.VMEM((_W, 128), jnp.float32),
            pltpu.VMEM((_W, D), jnp.float32),
            pltpu.VMEM((_W, D), jnp.float32),
            pltpu.SemaphoreType.DMA,
            pltpu.SemaphoreType.DMA,
            pltpu.SemaphoreType.DMA,
            pltpu.SemaphoreType.DMA,
        ],
    )
    def _gather(tab_hbm, idx_hbm, out_hbm, idx_all, buf0, buf1, ob0, ob1,
                sg0, sg1, sw0, sw1):
        wid = lax.axis_index("s") * _NC + lax.axis_index("c")
        base = wid * n_per
        b0 = wid * b_per
        pltpu.sync_copy(idx_hbm.at[pl.ds(base, n_per)], idx_all)

        def gather(c, buf, sem):
            pltpu.async_copy(tab_hbm.at[idx_all.at[pl.ds(c * _W, _W)]], buf, sem)

        def wait_gather(buf, sem):
            pltpu.make_async_copy(
                tab_hbm.at[idx_all.at[pl.ds(0, _W)]], buf, sem).wait()

        def repack(buf, ob):
            @pl.loop(0, _W)
            def _(r):
                for k in range(D // 16):
                    ob.at[pl.ds(r, 1), pl.ds(16 * k, 16)][...] = (
                        buf.at[pl.ds(r, 1), pl.ds(16 * k, 16)][...])

        def write(c, ob, sem):
            bc = b0 + c * bpc
            for j in range(bpc):
                pltpu.async_copy(ob.at[pl.ds(Th * j, Th)], out_hbm.at[bc + j], sem)

        def wait_write(ob, sem):
            for j in range(bpc):
                pltpu.make_async_copy(ob.at[pl.ds(Th * j, Th)], out_hbm.at[0], sem).wait()

        gather(0, buf0, sg0)
        gather(1, buf1, sg1)

        @pl.loop(0, n_chunks // 2)
        def _(i):
            c = 2 * i
            wait_gather(buf0, sg0)

            @pl.when(i > 0)
            def _():
                wait_write(ob0, sw0)

            repack(buf0, ob0)

            @pl.when(i < n_chunks // 2 - 1)
            def _():
                gather(c + 2, buf0, sg0)

            write(c, ob0, sw0)

            wait_gather(buf1, sg1)

            @pl.when(i > 0)
            def _():
                wait_write(ob1, sw1)

            repack(buf1, ob1)

            @pl.when(i < n_chunks // 2 - 1)
            def _():
                gather(c + 3, buf1, sg1)

            write(c + 1, ob1, sw1)

        wait_write(ob0, sw0)
        wait_write(ob1, sw1)

    return _gather


def kernel(x, table):
    B, T = x.shape
    V, D = table.shape
    big = jnp.pad(table, ((0, 0), (0, 128 - D)))
    Th = T // _K
    gather_half = _make_gather(B, Th, D, V)

    halves = []
    for k in range(_K):
        idxk = x[:, Th * k:Th * (k + 1)].reshape(-1).astype(jnp.int32)
        yk = gather_half(big, idxk)
        halves.append(jnp.transpose(yk, (1, 2, 0)))

    out_tdb = jnp.concatenate(halves, axis=0)
    return jnp.transpose(out_tdb, (2, 0, 1))
